# baseline (speedup 1.0000x reference)
import jax
import jax.numpy as jnp
from jax import lax
from jax.experimental import pallas as pl
from jax.experimental.pallas import tpu as pltpu
from jax.experimental.pallas import tpu_sc as plsc

_B = 16384
_L = 64
_D = 64
_V = 1000001

_NC = 2
_NS = 16
_NW = _NC * _NS
_BPW = _B // _NW
_GB = 4
_GROWS = _GB * _L
_NG = _BPW // _GB

_CK = 8192
_CH = _CK // 2
_G = (_V + _CK - 1) // _CK
_N2 = _G * _CK


def _permute_ids(j):
    c = j & (_CK - 1)
    return (j - c) + 2 * (c & (_CH - 1)) + (c >> 12)


def _transpose_body(x_ref, o_ref):
    x = x_ref[...]
    r = lax.broadcasted_iota(jnp.int32, (_D, _D), 0)
    c = lax.broadcasted_iota(jnp.int32, (_D, _D), 1)
    e = (r == c).astype(jnp.float32)
    xt = lax.dot_general(x, e, (((0,), (0,)), ((), ())),
                         preferred_element_type=jnp.float32)
    o_ref[:, 0:_D] = xt[0:_CH]
    o_ref[:, _D:2 * _D] = xt[_CH:_CK]


_transpose = pl.pallas_call(
    _transpose_body,
    grid=(_G,),
    in_specs=[pl.BlockSpec((_D, _CK), lambda i: (0, i))],
    out_specs=pl.BlockSpec((_CH, 2 * _D), lambda i: (i, 0)),
    out_shape=jax.ShapeDtypeStruct((_G * _CH, 2 * _D), jnp.float32),
)


def _sc_body(seq_hbm, last_hbm, table_hbm, sums_hbm, xt_hbm,
             idx_v, rows0, rows1, sum_v, lastid_v, sem0, sem1):
    wid = lax.axis_index("s") * _NC + lax.axis_index("c")
    base = wid * _BPW

    pltpu.sync_copy(seq_hbm.at[pl.ds(base * _L, _BPW * _L)], idx_v)
    pltpu.sync_copy(last_hbm.at[pl.ds(base, _BPW)], lastid_v)

    for k in range(_BPW // _GROWS):
        pltpu.async_copy(
            table_hbm.at[lastid_v.at[pl.ds(k * _GROWS, _GROWS)]], rows0,
            sem0).wait()
        pltpu.sync_copy(rows0, xt_hbm.at[pl.ds(base + k * _GROWS, _GROWS)])

    def start(g, buf, sem):
        pltpu.async_copy(
            table_hbm.at[idx_v.at[pl.ds(g * _GROWS, _GROWS)]], buf, sem)

    def wait(g, buf, sem):
        pltpu.make_async_copy(
            table_hbm.at[idx_v.at[pl.ds(g * _GROWS, _GROWS)]], buf, sem).wait()

    def reduce_buf(buf, g):
        for b in range(_GB):
            def red_step(l2, accs):
                r = b * _L + l2 * 4
                a = accs
                for u in range(4):
                    a = tuple(a[j] + buf[r + u, pl.ds(j * 16, 16)]
                              for j in range(4))
                return a
            accs = lax.fori_loop(
                0, _L // 4, red_step,
                tuple(jnp.zeros((16,), jnp.float32) for _ in range(4)))
            row = g * _GB + b
            for j in range(4):
                sum_v[row, pl.ds(j * 16, 16)] = accs[j]

    start(0, rows0, sem0)

    def body(h, carry):
        g0 = h * 2
        start(g0 + 1, rows1, sem1)
        wait(g0, rows0, sem0)
        reduce_buf(rows0, g0)

        @pl.when(g0 + 2 < _NG)
        def _():
            start(g0 + 2, rows0, sem0)

        wait(g0 + 1, rows1, sem1)
        reduce_buf(rows1, g0 + 1)
        return carry

    lax.fori_loop(0, _NG // 2, body, 0)

    pltpu.sync_copy(sum_v, sums_hbm.at[pl.ds(base, _BPW)])


_sc_pool = pl.kernel(
    _sc_body,
    out_type=(
        jax.ShapeDtypeStruct((_B, _D), jnp.float32),
        jax.ShapeDtypeStruct((_B, _D), jnp.float32),
    ),
    mesh=plsc.VectorSubcoreMesh(core_axis_name="c", subcore_axis_name="s",
                                num_cores=_NC, num_subcores=_NS),
    compiler_params=pltpu.CompilerParams(use_tc_tiling_on_sc=False),
    scratch_types=(
        pltpu.VMEM((_BPW * _L,), jnp.int32),
        pltpu.VMEM((_GROWS, _D), jnp.float32),
        pltpu.VMEM((_GROWS, _D), jnp.float32),
        pltpu.VMEM((_BPW, _D), jnp.float32),
        pltpu.VMEM((_BPW,), jnp.int32),
        pltpu.SemaphoreType.DMA,
        pltpu.SemaphoreType.DMA,
    ),
)


_BT = 2048


def _prep_body(seq_ref, cnt_ref, last_ref, pseq_ref):
    s = seq_ref[...]
    nz = jnp.where(s != 0, 1, 0)
    cnt = jnp.sum(nz, axis=1, keepdims=True)
    li = jnp.clip(cnt - 1, 0, _L - 1)
    pos = lax.broadcasted_iota(jnp.int32, (1, _L), 1)
    last = jnp.sum(jnp.where(pos == li, s, 0), axis=1, keepdims=True)
    cnt_ref[...] = cnt.astype(jnp.float32)
    last_ref[...] = _permute_ids(last)
    pseq_ref[...] = _permute_ids(s)


_prep = pl.pallas_call(
    _prep_body,
    grid=(_B // _BT,),
    in_specs=[pl.BlockSpec((_BT, _L), lambda i: (i, 0))],
    out_specs=[pl.BlockSpec((_BT, 1), lambda i: (i, 0)),
               pl.BlockSpec((_BT, 1), lambda i: (i, 0)),
               pl.BlockSpec((_BT, _L), lambda i: (i, 0))],
    out_shape=[jax.ShapeDtypeStruct((_B, 1), jnp.float32),
               jax.ShapeDtypeStruct((_B, 1), jnp.int32),
               jax.ShapeDtypeStruct((_B, _L), jnp.int32)],
)


def _finish_body(sums_ref, cnt_ref, xt_ref, wa_ref, ba_ref, wb_ref, bb_ref,
                 o_ref):
    m = sums_ref[...] / cnt_ref[...]
    hs = jnp.tanh(
        jnp.dot(m, wa_ref[...], preferred_element_type=jnp.float32)
        + ba_ref[...])
    ht = jnp.tanh(
        jnp.dot(xt_ref[...], wb_ref[...], preferred_element_type=jnp.float32)
        + bb_ref[...])
    o_ref[...] = hs * ht


_finish = pl.pallas_call(
    _finish_body,
    grid=(_B // _BT,),
    in_specs=[
        pl.BlockSpec((_BT, _D), lambda i: (i, 0)),
        pl.BlockSpec((_BT, 1), lambda i: (i, 0)),
        pl.BlockSpec((_BT, _D), lambda i: (i, 0)),
        pl.BlockSpec((_D, _D), lambda i: (0, 0)),
        pl.BlockSpec((1, _D), lambda i: (0, 0)),
        pl.BlockSpec((_D, _D), lambda i: (0, 0)),
        pl.BlockSpec((1, _D), lambda i: (0, 0)),
    ],
    out_specs=pl.BlockSpec((_BT, _D), lambda i: (i, 0)),
    out_shape=jax.ShapeDtypeStruct((_B, _D), jnp.float32),
)


@jax.jit
def kernel(item_seq, table, Wa, ba, Wb, bb):
    seq = item_seq.astype(jnp.int32)
    counts, last_p, pseq = _prep(seq)
    t2 = _transpose(table.T)
    tbl_lin = t2.reshape(_N2, _D)
    sums, xt = _sc_pool(pseq.reshape(-1), last_p.reshape(-1), tbl_lin)
    out = _finish(sums, counts, xt,
                  Wa.T, ba.reshape(1, _D), Wb.T, bb.reshape(1, _D))
    return out


# speedup vs baseline: 1.0770x; 1.0770x over previous
import jax
import jax.numpy as jnp
from jax import lax
from jax.experimental import pallas as pl
from jax.experimental.pallas import tpu as pltpu
from jax.experimental.pallas import tpu_sc as plsc

_B = 16384
_L = 64
_D = 64
_V = 1000001

_NC = 2
_NS = 16
_NW = _NC * _NS
_BPW = _B // _NW
_GB = 4
_GROWS = _GB * _L
_NG = _BPW // _GB

_CK = 16384
_CH = _CK // 2
_G = (_V + _CK - 1) // _CK
_N2 = _G * _CK


def _permute_ids(j):
    c = j & (_CK - 1)
    return (j - c) + 2 * (c & (_CH - 1)) + (c >> 13)


def _transpose_body(x_ref, o_ref):
    xt = x_ref[...].T
    o_ref[:, 0:_D] = xt[0:_CH]
    o_ref[:, _D:2 * _D] = xt[_CH:_CK]


_transpose = pl.pallas_call(
    _transpose_body,
    grid=(_G,),
    in_specs=[pl.BlockSpec((_D, _CK), lambda i: (0, i))],
    out_specs=pl.BlockSpec((_CH, 2 * _D), lambda i: (i, 0)),
    out_shape=jax.ShapeDtypeStruct((_G * _CH, 2 * _D), jnp.float32),
)


def _sc_body(seq_hbm, last_hbm, table_hbm, sums_hbm, xt_hbm,
             idx_v, rows0, rows1, sum_v, lastid_v, sem0, sem1):
    wid = lax.axis_index("s") * _NC + lax.axis_index("c")
    base = wid * _BPW

    pltpu.sync_copy(seq_hbm.at[pl.ds(base * _L, _BPW * _L)], idx_v)
    pltpu.sync_copy(last_hbm.at[pl.ds(base, _BPW)], lastid_v)

    for k in range(_BPW // _GROWS):
        pltpu.async_copy(
            table_hbm.at[lastid_v.at[pl.ds(k * _GROWS, _GROWS)]], rows0,
            sem0).wait()
        pltpu.sync_copy(rows0, xt_hbm.at[pl.ds(base + k * _GROWS, _GROWS)])

    def start(g, buf, sem):
        pltpu.async_copy(
            table_hbm.at[idx_v.at[pl.ds(g * _GROWS, _GROWS)]], buf, sem)

    def wait(g, buf, sem):
        pltpu.make_async_copy(
            table_hbm.at[idx_v.at[pl.ds(g * _GROWS, _GROWS)]], buf, sem).wait()

    def reduce_buf(buf, g):
        for b in range(_GB):
            def red_step(l2, accs):
                r = b * _L + l2 * 4
                a = accs
                for u in range(4):
                    a = tuple(a[j] + buf[r + u, pl.ds(j * 16, 16)]
                              for j in range(4))
                return a
            accs = lax.fori_loop(
                0, _L // 4, red_step,
                tuple(jnp.zeros((16,), jnp.float32) for _ in range(4)))
            row = g * _GB + b
            for j in range(4):
                sum_v[row, pl.ds(j * 16, 16)] = accs[j]

    start(0, rows0, sem0)

    def body(h, carry):
        g0 = h * 2
        start(g0 + 1, rows1, sem1)
        wait(g0, rows0, sem0)
        reduce_buf(rows0, g0)

        @pl.when(g0 + 2 < _NG)
        def _():
            start(g0 + 2, rows0, sem0)

        wait(g0 + 1, rows1, sem1)
        reduce_buf(rows1, g0 + 1)
        return carry

    lax.fori_loop(0, _NG // 2, body, 0)

    pltpu.sync_copy(sum_v, sums_hbm.at[pl.ds(base, _BPW)])


_sc_pool = pl.kernel(
    _sc_body,
    out_type=(
        jax.ShapeDtypeStruct((_B, _D), jnp.float32),
        jax.ShapeDtypeStruct((_B, _D), jnp.float32),
    ),
    mesh=plsc.VectorSubcoreMesh(core_axis_name="c", subcore_axis_name="s",
                                num_cores=_NC, num_subcores=_NS),
    compiler_params=pltpu.CompilerParams(use_tc_tiling_on_sc=False),
    scratch_types=(
        pltpu.VMEM((_BPW * _L,), jnp.int32),
        pltpu.VMEM((_GROWS, _D), jnp.float32),
        pltpu.VMEM((_GROWS, _D), jnp.float32),
        pltpu.VMEM((_BPW, _D), jnp.float32),
        pltpu.VMEM((_BPW,), jnp.int32),
        pltpu.SemaphoreType.DMA,
        pltpu.SemaphoreType.DMA,
    ),
)


_BT = 2048


def _prep_body(seq_ref, cnt_ref, last_ref, pseq_ref):
    s = seq_ref[...]
    nz = jnp.where(s != 0, 1, 0)
    cnt = jnp.sum(nz, axis=1, keepdims=True)
    li = jnp.clip(cnt - 1, 0, _L - 1)
    pos = lax.broadcasted_iota(jnp.int32, (1, _L), 1)
    last = jnp.sum(jnp.where(pos == li, s, 0), axis=1, keepdims=True)
    cnt_ref[...] = cnt.astype(jnp.float32)
    last_ref[...] = _permute_ids(last)
    pseq_ref[...] = _permute_ids(s)


_prep = pl.pallas_call(
    _prep_body,
    grid=(_B // _BT,),
    in_specs=[pl.BlockSpec((_BT, _L), lambda i: (i, 0))],
    out_specs=[pl.BlockSpec((_BT, 1), lambda i: (i, 0)),
               pl.BlockSpec((_BT, 1), lambda i: (i, 0)),
               pl.BlockSpec((_BT, _L), lambda i: (i, 0))],
    out_shape=[jax.ShapeDtypeStruct((_B, 1), jnp.float32),
               jax.ShapeDtypeStruct((_B, 1), jnp.int32),
               jax.ShapeDtypeStruct((_B, _L), jnp.int32)],
)


def _finish_body(sums_ref, cnt_ref, xt_ref, wa_ref, ba_ref, wb_ref, bb_ref,
                 o_ref):
    m = sums_ref[...] / cnt_ref[...]
    hs = jnp.tanh(
        jnp.dot(m, wa_ref[...], preferred_element_type=jnp.float32)
        + ba_ref[...])
    ht = jnp.tanh(
        jnp.dot(xt_ref[...], wb_ref[...], preferred_element_type=jnp.float32)
        + bb_ref[...])
    o_ref[...] = hs * ht


_finish = pl.pallas_call(
    _finish_body,
    grid=(_B // _BT,),
    in_specs=[
        pl.BlockSpec((_BT, _D), lambda i: (i, 0)),
        pl.BlockSpec((_BT, 1), lambda i: (i, 0)),
        pl.BlockSpec((_BT, _D), lambda i: (i, 0)),
        pl.BlockSpec((_D, _D), lambda i: (0, 0)),
        pl.BlockSpec((1, _D), lambda i: (0, 0)),
        pl.BlockSpec((_D, _D), lambda i: (0, 0)),
        pl.BlockSpec((1, _D), lambda i: (0, 0)),
    ],
    out_specs=pl.BlockSpec((_BT, _D), lambda i: (i, 0)),
    out_shape=jax.ShapeDtypeStruct((_B, _D), jnp.float32),
)


@jax.jit
def kernel(item_seq, table, Wa, ba, Wb, bb):
    seq = item_seq.astype(jnp.int32)
    counts, last_p, pseq = _prep(seq)
    t2 = _transpose(table.T)
    tbl_lin = t2.reshape(_N2, _D)
    sums, xt = _sc_pool(pseq.reshape(-1), last_p.reshape(-1), tbl_lin)
    out = _finish(sums, counts, xt,
                  Wa.T, ba.reshape(1, _D), Wb.T, bb.reshape(1, _D))
    return out


# speedup vs baseline: 1.1130x; 1.0334x over previous
import jax
import jax.numpy as jnp
from jax import lax
from jax.experimental import pallas as pl
from jax.experimental.pallas import tpu as pltpu
from jax.experimental.pallas import tpu_sc as plsc

_B = 16384
_L = 64
_D = 64
_V = 1000001

_NC = 2
_NS = 16
_NW = _NC * _NS
_BPW = _B // _NW
_GB = 4
_GROWS = _GB * _L
_NG = _BPW // _GB

_CK = 32768
_CH = _CK // 2
_G = (_V + _CK - 1) // _CK
_N2 = _G * _CK


def _permute_ids(j):
    c = j & (_CK - 1)
    return (j - c) + 2 * (c & (_CH - 1)) + (c >> 14)


def _transpose_body(x_ref, o_ref):
    xt = x_ref[...].T
    o_ref[:, 0:_D] = xt[0:_CH]
    o_ref[:, _D:2 * _D] = xt[_CH:_CK]


_transpose = pl.pallas_call(
    _transpose_body,
    grid=(_G,),
    in_specs=[pl.BlockSpec((_D, _CK), lambda i: (0, i))],
    out_specs=pl.BlockSpec((_CH, 2 * _D), lambda i: (i, 0)),
    out_shape=jax.ShapeDtypeStruct((_G * _CH, 2 * _D), jnp.float32),
)


def _sc_body(seq_hbm, last_hbm, table_hbm, sums_hbm, xt_hbm,
             idx_v, rows0, rows1, sum_v, lastid_v, sem0, sem1):
    wid = lax.axis_index("s") * _NC + lax.axis_index("c")
    base = wid * _BPW

    pltpu.sync_copy(seq_hbm.at[pl.ds(base * _L, _BPW * _L)], idx_v)
    pltpu.sync_copy(last_hbm.at[pl.ds(base, _BPW)], lastid_v)

    for k in range(_BPW // _GROWS):
        pltpu.async_copy(
            table_hbm.at[lastid_v.at[pl.ds(k * _GROWS, _GROWS)]], rows0,
            sem0).wait()
        pltpu.sync_copy(rows0, xt_hbm.at[pl.ds(base + k * _GROWS, _GROWS)])

    def start(g, buf, sem):
        pltpu.async_copy(
            table_hbm.at[idx_v.at[pl.ds(g * _GROWS, _GROWS)]], buf, sem)

    def wait(g, buf, sem):
        pltpu.make_async_copy(
            table_hbm.at[idx_v.at[pl.ds(g * _GROWS, _GROWS)]], buf, sem).wait()

    def reduce_buf(buf, g):
        for b in range(_GB):
            def red_step(l2, accs):
                r = b * _L + l2 * 4
                a = accs
                for u in range(4):
                    a = tuple(a[j] + buf[r + u, pl.ds(j * 16, 16)]
                              for j in range(4))
                return a
            accs = lax.fori_loop(
                0, _L // 4, red_step,
                tuple(jnp.zeros((16,), jnp.float32) for _ in range(4)))
            row = g * _GB + b
            for j in range(4):
                sum_v[row, pl.ds(j * 16, 16)] = accs[j]

    start(0, rows0, sem0)

    def body(h, carry):
        g0 = h * 2
        start(g0 + 1, rows1, sem1)
        wait(g0, rows0, sem0)
        reduce_buf(rows0, g0)

        @pl.when(g0 + 2 < _NG)
        def _():
            start(g0 + 2, rows0, sem0)

        wait(g0 + 1, rows1, sem1)
        reduce_buf(rows1, g0 + 1)
        return carry

    lax.fori_loop(0, _NG // 2, body, 0)

    pltpu.sync_copy(sum_v, sums_hbm.at[pl.ds(base, _BPW)])


_sc_pool = pl.kernel(
    _sc_body,
    out_type=(
        jax.ShapeDtypeStruct((_B, _D), jnp.float32),
        jax.ShapeDtypeStruct((_B, _D), jnp.float32),
    ),
    mesh=plsc.VectorSubcoreMesh(core_axis_name="c", subcore_axis_name="s",
                                num_cores=_NC, num_subcores=_NS),
    compiler_params=pltpu.CompilerParams(use_tc_tiling_on_sc=False),
    scratch_types=(
        pltpu.VMEM((_BPW * _L,), jnp.int32),
        pltpu.VMEM((_GROWS, _D), jnp.float32),
        pltpu.VMEM((_GROWS, _D), jnp.float32),
        pltpu.VMEM((_BPW, _D), jnp.float32),
        pltpu.VMEM((_BPW,), jnp.int32),
        pltpu.SemaphoreType.DMA,
        pltpu.SemaphoreType.DMA,
    ),
)


_BT = 2048


def _prep_body(seq_ref, cnt_ref, last_ref, pseq_ref):
    s = seq_ref[...]
    nz = jnp.where(s != 0, 1, 0)
    cnt = jnp.sum(nz, axis=1, keepdims=True)
    li = jnp.clip(cnt - 1, 0, _L - 1)
    pos = lax.broadcasted_iota(jnp.int32, (1, _L), 1)
    last = jnp.sum(jnp.where(pos == li, s, 0), axis=1, keepdims=True)
    cnt_ref[...] = cnt.astype(jnp.float32)
    last_ref[...] = _permute_ids(last)
    pseq_ref[...] = _permute_ids(s)


_prep = pl.pallas_call(
    _prep_body,
    grid=(_B // _BT,),
    in_specs=[pl.BlockSpec((_BT, _L), lambda i: (i, 0))],
    out_specs=[pl.BlockSpec((_BT, 1), lambda i: (i, 0)),
               pl.BlockSpec((_BT, 1), lambda i: (i, 0)),
               pl.BlockSpec((_BT, _L), lambda i: (i, 0))],
    out_shape=[jax.ShapeDtypeStruct((_B, 1), jnp.float32),
               jax.ShapeDtypeStruct((_B, 1), jnp.int32),
               jax.ShapeDtypeStruct((_B, _L), jnp.int32)],
)


def _finish_body(sums_ref, cnt_ref, xt_ref, wa_ref, ba_ref, wb_ref, bb_ref,
                 o_ref):
    m = sums_ref[...] / cnt_ref[...]
    hs = jnp.tanh(
        jnp.dot(m, wa_ref[...], preferred_element_type=jnp.float32)
        + ba_ref[...])
    ht = jnp.tanh(
        jnp.dot(xt_ref[...], wb_ref[...], preferred_element_type=jnp.float32)
        + bb_ref[...])
    o_ref[...] = hs * ht


_finish = pl.pallas_call(
    _finish_body,
    grid=(_B // _BT,),
    in_specs=[
        pl.BlockSpec((_BT, _D), lambda i: (i, 0)),
        pl.BlockSpec((_BT, 1), lambda i: (i, 0)),
        pl.BlockSpec((_BT, _D), lambda i: (i, 0)),
        pl.BlockSpec((_D, _D), lambda i: (0, 0)),
        pl.BlockSpec((1, _D), lambda i: (0, 0)),
        pl.BlockSpec((_D, _D), lambda i: (0, 0)),
        pl.BlockSpec((1, _D), lambda i: (0, 0)),
    ],
    out_specs=pl.BlockSpec((_BT, _D), lambda i: (i, 0)),
    out_shape=jax.ShapeDtypeStruct((_B, _D), jnp.float32),
)


@jax.jit
def kernel(item_seq, table, Wa, ba, Wb, bb):
    seq = item_seq.astype(jnp.int32)
    counts, last_p, pseq = _prep(seq)
    t2 = _transpose(table.T)
    tbl_lin = t2.reshape(_N2, _D)
    sums, xt = _sc_pool(pseq.reshape(-1), last_p.reshape(-1), tbl_lin)
    out = _finish(sums, counts, xt,
                  Wa.T, ba.reshape(1, _D), Wb.T, bb.reshape(1, _D))
    return out
